# baseline (speedup 1.0000x reference)
import dataclasses

import jax
import jax.numpy as jnp
from jax import lax
from jax.experimental import pallas as pl
from jax.experimental.pallas import tpu as pltpu
from jax.experimental.pallas import tpu_sc as plsc

N = 10000
E = 160000
IN_F = 256
OUT_F = 64
H = 8
HID = H * OUT_F
NQ = 4
QH = 2
QF = QH * OUT_F

NT = 32
RNG = 320
BASE_ROWS = N - RNG * (NT - 1)
AGG_ROWS = 336
JUNK = AGG_ROWS - 1
CAP = 6144
FB = 2000
NBLK = E // FB
B4 = 128
NSLOT = 2
NB4 = CAP // B4

_BLK = 400
_PREC = lax.Precision.HIGHEST
_f32 = jnp.float32


def _proj_body(x_ref, wv_ref, bv_ref, wq_ref, bq_ref, wk_ref, bk_ref, *outs):
    h = jax.lax.dot_general(x_ref[...], wv_ref[...], (((1,), (0,)), ((), ())),
                            precision=_PREC, preferred_element_type=_f32)
    h = h + bv_ref[...]
    q = jax.lax.dot_general(h, wq_ref[...], (((1,), (0,)), ((), ())),
                            precision=_PREC, preferred_element_type=_f32)
    q = q + bq_ref[...]
    k = jax.lax.dot_general(h, wk_ref[...], (((1,), (0,)), ((), ())),
                            precision=_PREC, preferred_element_type=_f32)
    k = k + bk_ref[...]
    for i in range(NQ):
        outs[i][...] = h[:, i * QF:(i + 1) * QF]
        outs[NQ + i][...] = q[:, i * QH:(i + 1) * QH]
        outs[2 * NQ + i][...] = k[:, i * QH:(i + 1) * QH]


def _proj(x, Wv, bv, Wq, bq, Wk, bk):
    full = lambda s: pl.BlockSpec(s, lambda i: tuple(0 for _ in s))
    row = lambda c: pl.BlockSpec((_BLK, c), lambda i: (i, 0))
    return pl.pallas_call(
        _proj_body,
        grid=(N // _BLK,),
        in_specs=[row(IN_F), full((IN_F, HID)), full((1, HID)),
                  full((HID, H)), full((1, H)), full((HID, H)), full((1, H))],
        out_specs=([row(QF)] * NQ + [row(QH)] * (2 * NQ)),
        out_shape=([jax.ShapeDtypeStruct((N, QF), _f32)] * NQ
                   + [jax.ShapeDtypeStruct((N, QH), _f32)] * (2 * NQ)),
    )(x, Wv, bv, Wq, bq, Wk, bk)


def _sc_body(esrc_ref, edst_ref, *rest):
    h_t = rest[0:NQ]
    q_t = rest[NQ:2 * NQ]
    k_t = rest[2 * NQ:3 * NQ]
    a_t = rest[3 * NQ:4 * NQ]
    (kept_src, kept_dst, sbs0, sbs1, sbd0, sbd1, q_loc, k_loc, s_loc,
     exb, hb0, hb1, agg, kept_src2,
     semf, semh, semm) = rest[4 * NQ:]
    sb_src = (sbs0, sbs1)
    sb_dst = (sbd0, sbd1)
    hb = (hb0, hb1)

    wid = lax.axis_index("s") * 2 + lax.axis_index("c")
    lo = wid * RNG
    i16 = lax.iota(jnp.int32, 16)
    zf16 = jnp.zeros((16,), _f32)
    e2 = i16 // 2
    h2 = i16 - 2 * e2

    @pl.loop(0, CAP + 16, step=16)
    def _(i):
        kept_dst[pl.ds(i, 16)] = jnp.full((16,), JUNK, jnp.int32)
        kept_src[pl.ds(i, 16)] = jnp.zeros((16,), jnp.int32)

    def _start_blk(blk, sub):
        pltpu.make_async_copy(esrc_ref.at[pl.ds(blk * FB, FB)],
                              sb_src[sub], semf.at[2 * sub]).start()
        pltpu.make_async_copy(edst_ref.at[pl.ds(blk * FB, FB)],
                              sb_dst[sub], semf.at[2 * sub + 1]).start()

    _start_blk(0, 0)
    _start_blk(1, 1)

    def _filter_blk(g, sub, C):
        blk = 2 * g + sub
        pltpu.make_async_copy(esrc_ref.at[pl.ds(blk * FB, FB)],
                              sb_src[sub], semf.at[2 * sub]).wait()
        pltpu.make_async_copy(edst_ref.at[pl.ds(blk * FB, FB)],
                              sb_dst[sub], semf.at[2 * sub + 1]).wait()

        def _vreg(j, C):
            d = sb_dst[sub][pl.ds(j * 16, 16)]
            s = sb_src[sub][pl.ds(j * 16, 16)]
            rel = d - lo
            m = (rel >= 0) & (rel < RNG)
            plsc.store_compressed(kept_dst.at[pl.ds(C, 16)], rel, mask=m)
            plsc.store_compressed(kept_src.at[pl.ds(C, 16)], s, mask=m)
            cnt = plsc.all_reduce_population_count(m)
            return jnp.minimum(C + cnt[0], CAP)

        C = lax.fori_loop(0, FB // 16, _vreg, C)

        @pl.when(blk + 2 < NBLK)
        def _():
            _start_blk(blk + 2, sub)

        return C

    def _filter_pair(g, C):
        C = _filter_blk(g, 0, C)
        C = _filter_blk(g, 1, C)
        return C

    lax.fori_loop(0, NBLK // 2, _filter_pair, jnp.int32(0))

    @pl.loop(0, NB4)
    def _(b):
        @pl.loop(0, B4, step=16)
        def _(c):
            kept_src2[b, pl.ds(c, 16)] = kept_src[pl.ds(b * B4 + c, 16)] * 0

    for qtr in range(NQ):
        ht = h_t[qtr]
        at = a_t[qtr]

        pltpu.async_copy(q_t[qtr], q_loc, semm).wait()
        pltpu.async_copy(k_t[qtr].at[pl.ds(lo * QH, BASE_ROWS * QH)],
                         k_loc.at[pl.ds(0, BASE_ROWS * QH)], semm).wait()

        @pl.when(lo + RNG <= N)
        def _():
            pltpu.async_copy(
                k_t[qtr].at[pl.ds((lo + BASE_ROWS) * QH,
                                  (RNG - BASE_ROWS) * QH)],
                k_loc.at[pl.ds(BASE_ROWS * QH, (RNG - BASE_ROWS) * QH)],
                semm).wait()

        @pl.loop(0, AGG_ROWS * QH, step=16)
        def _(i):
            s_loc[pl.ds(i, 16)] = zf16

        @pl.loop(0, AGG_ROWS, step=1)
        def _(r):
            @pl.loop(0, QF, step=16)
            def _(c):
                agg[r, pl.ds(c, 16)] = zf16

        def _ex_vreg(base8):
            erow = base8 + e2
            rel = plsc.load_gather(kept_dst, [erow])
            srcv = plsc.load_gather(kept_src, [erow])
            qv = plsc.load_gather(q_loc, [srcv * QH + h2])
            kv = plsc.load_gather(k_loc, [rel * QH + h2])
            ev = qv + kv
            co = jnp.maximum(ev, 0.2 * ev)
            return rel, jnp.exp(co)

        @plsc.parallel_loop(0, CAP // 8, unroll=4)
        def _(j):
            rel, ex = _ex_vreg(j * 8)
            plsc.addupdate_scatter(s_loc, [rel * QH + h2], ex)

        def _b4_start(b, slot):
            pltpu.make_async_copy(ht.at[kept_src2.at[b]], hb[slot],
                                  semh.at[slot]).start()

        def _b4_wait(b, slot):
            pltpu.make_async_copy(ht.at[kept_src2.at[b]], hb[slot],
                                  semh.at[slot]).wait()

        for s in range(NSLOT):
            _b4_start(s, s)

        def _a_batch(b, slot):
            _b4_wait(b, slot)

            @plsc.parallel_loop(0, B4 // 8, unroll=2)
            def _(j):
                rel, ex = _ex_vreg(b * B4 + j * 8)
                den = plsc.load_gather(s_loc, [rel * QH + h2])
                exb[pl.ds(j * 16, 16)] = ex / den

            @plsc.parallel_loop(0, B4, unroll=2)
            def _(i):
                isp = jnp.zeros((16,), jnp.int32) + i
                rel = plsc.load_gather(kept_dst, [b * B4 + isp])
                a0 = plsc.load_gather(exb, [isp * 2])
                a1 = plsc.load_gather(exb, [isp * 2 + 1])
                for c in range(QF // 16):
                    att = a0 if c < 4 else a1
                    v = hb[slot][i, pl.ds(c * 16, 16)] * att
                    plsc.addupdate_scatter(agg, [rel, c * 16 + i16], v)

            @pl.when(b + NSLOT < NB4)
            def _():
                _b4_start(b + NSLOT, slot)

        def _a_group(g, _):
            for s in range(NSLOT):
                _a_batch(NSLOT * g + s, s)
            return 0

        lax.fori_loop(0, NB4 // NSLOT, _a_group, 0)

        pltpu.sync_copy(agg.at[pl.ds(0, BASE_ROWS)],
                        at.at[pl.ds(lo, BASE_ROWS)])

        @pl.when(lo + RNG <= N)
        def _():
            pltpu.sync_copy(agg.at[pl.ds(BASE_ROWS, RNG - BASE_ROWS)],
                            at.at[pl.ds(lo + BASE_ROWS, RNG - BASE_ROWS)])


def _sc_gat(esrc, edst, h_q, q_q, k_q):
    mesh = plsc.VectorSubcoreMesh(core_axis_name="c", subcore_axis_name="s")
    i32 = jnp.int32
    cp = pltpu.CompilerParams()
    if "needs_layout_passes" in pltpu.CompilerParams.__dataclass_fields__:
        cp = dataclasses.replace(cp, needs_layout_passes=False)
    kern = pl.kernel(
        _sc_body,
        out_type=tuple(jax.ShapeDtypeStruct((N, QF), _f32) for _ in range(NQ)),
        mesh=mesh,
        scratch_types=[
            pltpu.VMEM((CAP + 16,), i32),
            pltpu.VMEM((CAP + 16,), i32),
            pltpu.VMEM((FB,), i32),
            pltpu.VMEM((FB,), i32),
            pltpu.VMEM((FB,), i32),
            pltpu.VMEM((FB,), i32),
            pltpu.VMEM((N * QH,), _f32),
            pltpu.VMEM((AGG_ROWS * QH,), _f32),
            pltpu.VMEM((AGG_ROWS * QH,), _f32),
            pltpu.VMEM((B4 * QH,), _f32),
            pltpu.VMEM((B4, QF), _f32),
            pltpu.VMEM((B4, QF), _f32),
            pltpu.VMEM((AGG_ROWS, QF), _f32),
            pltpu.VMEM((NB4, B4), i32),
            pltpu.SemaphoreType.DMA((4,)),
            pltpu.SemaphoreType.DMA((NSLOT,)),
            pltpu.SemaphoreType.DMA,
        ],
        compiler_params=cp,
    )
    return kern(esrc, edst, *h_q, *q_q, *k_q)


def _mean_body(a0_ref, a1_ref, a2_ref, a3_ref, out_ref):
    acc = a0_ref[:, :OUT_F]
    for r in (a0_ref, a1_ref, a2_ref, a3_ref):
        acc = acc + r[:, OUT_F:]
        if r is not a0_ref:
            acc = acc + r[:, :OUT_F]
    out_ref[...] = acc * (1.0 / H)


def _mean(aggs):
    return pl.pallas_call(
        _mean_body,
        grid=(N // _BLK,),
        in_specs=[pl.BlockSpec((_BLK, QF), lambda i: (i, 0))] * NQ,
        out_specs=pl.BlockSpec((_BLK, OUT_F), lambda i: (i, 0)),
        out_shape=jax.ShapeDtypeStruct((N, OUT_F), _f32),
    )(*aggs)


def kernel(x, edge_index, Wv, bv, Wq, bq, Wk, bk):
    outs = _proj(x, Wv, bv.reshape(1, HID), Wq, bq.reshape(1, H),
                 Wk, bk.reshape(1, H))
    h_q, q_q, k_q = outs[:NQ], outs[NQ:2 * NQ], outs[2 * NQ:]
    q_q = [q.reshape(N * QH) for q in q_q]
    k_q = [k.reshape(N * QH) for k in k_q]
    aggs = _sc_gat(edge_index[0], edge_index[1], h_q, q_q, k_q)
    return _mean(aggs)


# speedup vs baseline: 11.9236x; 11.9236x over previous
import dataclasses

import jax
import jax.numpy as jnp
from jax import lax
from jax.experimental import pallas as pl
from jax.experimental.pallas import tpu as pltpu
from jax.experimental.pallas import tpu_sc as plsc

N = 10000
E = 160000
IN_F = 256
OUT_F = 64
H = 8
HID = H * OUT_F
HEXT = 640
QOFF = 512

NT = 32
NR = 96
RNG = 112
LROWS = 120
JUNK = LROWS - 1
TAIL = N - RNG * 89
CAP = 2176
FB = 2000
NBLK = E // FB
B4 = 32
NB4 = CAP // B4

_BLK = 400
_PREC = lax.Precision.HIGHEST
_f32 = jnp.float32


def _proj_body(x_ref, wv_ref, bv_ref, wq_ref, bq_ref, wk_ref, bk_ref,
               hext_ref, k2_ref):
    h = jax.lax.dot_general(x_ref[...], wv_ref[...], (((1,), (0,)), ((), ())),
                            precision=_PREC, preferred_element_type=_f32)
    h = h + bv_ref[...]
    q = jax.lax.dot_general(h, wq_ref[...], (((1,), (0,)), ((), ())),
                            precision=_PREC, preferred_element_type=_f32)
    q = q + bq_ref[...]
    k = jax.lax.dot_general(h, wk_ref[...], (((1,), (0,)), ((), ())),
                            precision=_PREC, preferred_element_type=_f32)
    k = k + bk_ref[...]
    hext_ref[:, :HID] = h
    hext_ref[:, HID:HID + H] = q
    hext_ref[:, HID + H:HID + 2 * H] = k
    hext_ref[:, HID + 2 * H:] = jnp.zeros((_BLK, HEXT - HID - 2 * H), _f32)
    k2_ref[...] = k


def _proj(x, Wv, bv, Wq, bq, Wk, bk):
    full = lambda s: pl.BlockSpec(s, lambda i: tuple(0 for _ in s))
    row = lambda c: pl.BlockSpec((_BLK, c), lambda i: (i, 0))
    return pl.pallas_call(
        _proj_body,
        grid=(N // _BLK,),
        in_specs=[row(IN_F), full((IN_F, HID)), full((1, HID)),
                  full((HID, H)), full((1, H)), full((HID, H)), full((1, H))],
        out_specs=[row(HEXT), row(H)],
        out_shape=[jax.ShapeDtypeStruct((N, HEXT), _f32),
                   jax.ShapeDtypeStruct((N, H), _f32)],
    )(x, Wv, bv, Wq, bq, Wk, bk)


def _sc_body(esrc_ref, edst_ref, hext_ref, kflat_ref, out_ref,
             ks0, kd0, ks1, kd1, ks2, kd2, sbs0, sbs1, sbd0, sbd1,
             k_loc, s_loc, exb, hb0, hb1, agg, semf, semh, semm):
    kept_src = (ks0, ks1, ks2)
    kept_dst = (kd0, kd1, kd2)
    sb_src = (sbs0, sbs1)
    sb_dst = (sbd0, sbd1)
    hb = (hb0, hb1)

    wid = lax.axis_index("s") * 2 + lax.axis_index("c")
    i16 = lax.iota(jnp.int32, 16)
    zf16 = jnp.zeros((16,), _f32)
    e8 = i16 // 8
    h8 = i16 - 8 * e8

    for rr in range(3):
        @pl.loop(0, CAP + 16, step=16)
        def _(i, _rr=rr):
            kept_dst[_rr][pl.ds(i, 16)] = jnp.full((16,), JUNK, jnp.int32)
            kept_src[_rr][pl.ds(i, 16)] = jnp.zeros((16,), jnp.int32)

    los = [(wid + 32 * rr) * RNG for rr in range(3)]

    def _start_blk(blk, sub):
        pltpu.make_async_copy(esrc_ref.at[pl.ds(blk * FB, FB)],
                              sb_src[sub], semf.at[2 * sub]).start()
        pltpu.make_async_copy(edst_ref.at[pl.ds(blk * FB, FB)],
                              sb_dst[sub], semf.at[2 * sub + 1]).start()

    _start_blk(0, 0)
    _start_blk(1, 1)

    def _filter_blk(g, sub, Cs):
        blk = 2 * g + sub
        pltpu.make_async_copy(esrc_ref.at[pl.ds(blk * FB, FB)],
                              sb_src[sub], semf.at[2 * sub]).wait()
        pltpu.make_async_copy(edst_ref.at[pl.ds(blk * FB, FB)],
                              sb_dst[sub], semf.at[2 * sub + 1]).wait()

        def _vreg(j, Cs):
            d = sb_dst[sub][pl.ds(j * 16, 16)]
            s = sb_src[sub][pl.ds(j * 16, 16)]
            out = []
            for rr in range(3):
                rel = d - los[rr]
                m = (rel >= 0) & (rel < RNG)
                plsc.store_compressed(kept_dst[rr].at[pl.ds(Cs[rr], 16)],
                                      rel, mask=m)
                plsc.store_compressed(kept_src[rr].at[pl.ds(Cs[rr], 16)],
                                      s, mask=m)
                cnt = plsc.all_reduce_population_count(m)
                out.append(jnp.minimum(Cs[rr] + cnt[0], CAP))
            return tuple(out)

        Cs = lax.fori_loop(0, FB // 16, _vreg, Cs)

        @pl.when(blk + 2 < NBLK)
        def _():
            _start_blk(blk + 2, sub)

        return Cs

    def _filter_pair(g, Cs):
        Cs = _filter_blk(g, 0, Cs)
        Cs = _filter_blk(g, 1, Cs)
        return Cs

    lax.fori_loop(0, NBLK // 2, _filter_pair,
                  (jnp.int32(0), jnp.int32(0), jnp.int32(0)))

    for rr in range(3):
        lo = los[rr]
        ksr = kept_src[rr]
        kdr = kept_dst[rr]

        @pl.when(lo + LROWS <= N)
        def _():
            pltpu.async_copy(kflat_ref.at[pl.ds(lo * H, LROWS * H)],
                             k_loc, semm).wait()

        @pl.when((lo < N) & (lo + LROWS > N))
        def _():
            pltpu.async_copy(kflat_ref.at[pl.ds(lo * H, TAIL * H)],
                             k_loc.at[pl.ds(0, TAIL * H)], semm).wait()

        @pl.loop(0, LROWS * H, step=16)
        def _(i):
            s_loc[pl.ds(i, 16)] = zf16

        @pl.loop(0, LROWS, step=1)
        def _(r):
            @pl.loop(0, HID, step=16)
            def _(c):
                agg[r, pl.ds(c, 16)] = zf16

        def _b4_start(b, slot):
            pltpu.make_async_copy(hext_ref.at[ksr.at[pl.ds(b * B4, B4)]],
                                  hb[slot], semh.at[slot]).start()

        def _b4_wait(b, slot):
            pltpu.make_async_copy(hext_ref.at[ksr.at[pl.ds(b * B4, B4)]],
                                  hb[slot], semh.at[slot]).wait()

        _b4_start(0, 0)
        _b4_start(1, 1)

        def _sweep_batch(b, slot):
            _b4_wait(b, slot)

            @plsc.parallel_loop(0, B4 // 2, unroll=2)
            def _(j):
                erow = b * B4 + j * 2 + e8
                rel = plsc.load_gather(kdr, [erow])
                qv = plsc.load_gather(hb[slot], [j * 2 + e8, QOFF + h8])
                kv = plsc.load_gather(k_loc, [rel * H + h8])
                ev = qv + kv
                ex = jnp.exp(jnp.maximum(ev, 0.2 * ev))
                plsc.addupdate_scatter(s_loc, [rel * H + h8], ex)
                exb[pl.ds(j * 16, 16)] = ex

            @plsc.parallel_loop(0, B4, unroll=2)
            def _(i):
                isp = jnp.zeros((16,), jnp.int32) + i
                rel = plsc.load_gather(kdr, [b * B4 + isp])
                att = [plsc.load_gather(exb, [isp * H + hh]) for hh in range(H)]
                for c in range(HID // 16):
                    v = hb[slot][i, pl.ds(c * 16, 16)] * att[c // 4]
                    plsc.addupdate_scatter(agg, [rel, c * 16 + i16], v)

            @pl.when(b + 2 < NB4)
            def _():
                _b4_start(b + 2, slot)

        def _sweep_pair(g, _):
            _sweep_batch(2 * g, 0)
            _sweep_batch(2 * g + 1, 1)
            return 0

        lax.fori_loop(0, NB4 // 2, _sweep_pair, 0)

        @pl.loop(0, RNG, step=1)
        def _(r):
            for c in range(HID // 16):
                den = plsc.load_gather(
                    s_loc, [jnp.zeros((16,), jnp.int32) + (r * H + c // 4)])
                agg[r, pl.ds(c * 16, 16)] = (
                    agg[r, pl.ds(c * 16, 16)] / jnp.maximum(den, 1e-30))

        @pl.when(lo + RNG <= N)
        def _():
            pltpu.sync_copy(agg.at[pl.ds(0, RNG)], out_ref.at[pl.ds(lo, RNG)])

        @pl.when((lo < N) & (lo + RNG > N))
        def _():
            pltpu.sync_copy(agg.at[pl.ds(0, TAIL)],
                            out_ref.at[pl.ds(lo, TAIL)])


def _sc_gat(esrc, edst, hext, kflat):
    mesh = plsc.VectorSubcoreMesh(core_axis_name="c", subcore_axis_name="s")
    i32 = jnp.int32
    cp = pltpu.CompilerParams()
    if "needs_layout_passes" in pltpu.CompilerParams.__dataclass_fields__:
        cp = dataclasses.replace(cp, needs_layout_passes=False)
    kern = pl.kernel(
        _sc_body,
        out_type=jax.ShapeDtypeStruct((N, HID), _f32),
        mesh=mesh,
        scratch_types=[
            pltpu.VMEM((CAP + 16,), i32),
            pltpu.VMEM((CAP + 16,), i32),
            pltpu.VMEM((CAP + 16,), i32),
            pltpu.VMEM((CAP + 16,), i32),
            pltpu.VMEM((CAP + 16,), i32),
            pltpu.VMEM((CAP + 16,), i32),
            pltpu.VMEM((FB,), i32),
            pltpu.VMEM((FB,), i32),
            pltpu.VMEM((FB,), i32),
            pltpu.VMEM((FB,), i32),
            pltpu.VMEM((LROWS * H,), _f32),
            pltpu.VMEM((LROWS * H,), _f32),
            pltpu.VMEM((B4 * H,), _f32),
            pltpu.VMEM((B4, HEXT), _f32),
            pltpu.VMEM((B4, HEXT), _f32),
            pltpu.VMEM((LROWS, HID), _f32),
            pltpu.SemaphoreType.DMA((4,)),
            pltpu.SemaphoreType.DMA((2,)),
            pltpu.SemaphoreType.DMA,
        ],
        compiler_params=cp,
    )
    return kern(esrc, edst, hext, kflat)


def _mean_body(a_ref, out_ref):
    acc = a_ref[:, :OUT_F]
    for j in range(1, H):
        acc = acc + a_ref[:, j * OUT_F:(j + 1) * OUT_F]
    out_ref[...] = acc * (1.0 / H)


def _mean(a):
    return pl.pallas_call(
        _mean_body,
        grid=(N // _BLK,),
        in_specs=[pl.BlockSpec((_BLK, HID), lambda i: (i, 0))],
        out_specs=pl.BlockSpec((_BLK, OUT_F), lambda i: (i, 0)),
        out_shape=jax.ShapeDtypeStruct((N, OUT_F), _f32),
    )(a)


def kernel(x, edge_index, Wv, bv, Wq, bq, Wk, bk):
    hext, k2 = _proj(x, Wv, bv.reshape(1, HID), Wq, bq.reshape(1, H),
                     Wk, bk.reshape(1, H))
    agg = _sc_gat(edge_index[0], edge_index[1], hext, k2.reshape(N * H))
    return _mean(agg)
